# Initial kernel scaffold; baseline (speedup 1.0000x reference)
import jax
import jax.numpy as jnp
from jax.experimental import pallas as pl


def kernel(inp_node_features, inp_adj_features, noise_node, noise_edge, rgcn_Wself0, rgcn_Wrel0, rgcn_Wself1, rgcn_Wrel1, rgcn_Wself2, rgcn_Wrel2, node_Ws, node_bs, node_Wt, node_bt, edge_Ws, edge_bs, edge_Wt, edge_bt, rows, cols):
    raise NotImplementedError("write your pallas kernel here")


# speedup vs baseline: 3.3236x; 3.3236x over previous
import functools

import jax
import jax.numpy as jnp
from jax.experimental import pallas as pl
from jax.experimental.pallas import tpu as pltpu

B = 64
N = 128
ND = 16
BD = 3
NHID = 128
NOUT = 128
NFLOW = 12
DEQ = 0.9
E = 1458
EPAD = 1536


def _log_sigmoid(x):
    return jnp.minimum(x, 0.0) - jnp.log(1.0 + jnp.exp(-jnp.abs(x)))


def _fused_body(xT, adjT, nnT, neT, oct_ref, ort_ref,
                ws0, wr0, ws1, wr1, ws2, wr2,
                nws, nbs, nwt, nbt,
                estop, esbot, ettop, etbot, ebs, ebt,
                zn_ref, ze_ref, ldn_ref, lde_ref):
    f32 = jnp.float32
    dot = functools.partial(jnp.dot, preferred_element_type=f32)

    x = xT[0]
    adj = adjT[0]
    OCT = oct_ref[...]
    ORT = ort_ref[...]

    def rgcn(hT, WselfT, WrelT):
        acc = dot(WselfT, hT)
        for r in range(BD):
            inner = dot(hT, adj[r])
            acc = acc + dot(WrelT[r], inner)
        return jnp.maximum(acc, 0.0)

    h = rgcn(x, ws0[...], wr0[...])
    h = rgcn(h, ws1[...], wr1[...])
    h = rgcn(h, ws2[...], wr2[...])

    xc = x + DEQ * nnT[0]
    nws_v = nws[...]
    nwt_v = nwt[...]
    nbs_v = nbs[...]
    nbt_v = nbt[...]
    ldn = jnp.zeros((), f32)
    for l in range(NFLOW):
        pre = dot(nws_v[l], h) + nbs_v[l]
        s = jax.nn.sigmoid(pre)
        t = dot(nwt_v[l], h) + nbt_v[l]
        xc = xc * s + t
        ldn = ldn + jnp.sum(_log_sigmoid(pre))
    zn_ref[0] = xc
    ldn_ref[...] = jnp.zeros((1, 1, 128), f32) + ldn

    sels = []
    for r in range(BD):
        m = dot(adj[r], ORT)
        sels.append(jnp.sum(m * OCT, axis=0, keepdims=True))
    ec = jnp.concatenate(sels, axis=0) + DEQ * neT[0]

    pre_s = dot(dot(estop[...], h), OCT) + dot(dot(esbot[...], h), ORT)
    pre_t = dot(dot(ettop[...], h), OCT) + dot(dot(etbot[...], h), ORT)

    ebs_v = ebs[...]
    ebt_v = ebt[...]
    emask = jax.lax.broadcasted_iota(jnp.int32, (1, EPAD), 1) < E
    lde = jnp.zeros((), f32)
    for l in range(NFLOW):
        ps = pre_s[3 * l:3 * l + 3] + ebs_v[l]
        s = jax.nn.sigmoid(ps)
        t = pre_t[3 * l:3 * l + 3] + ebt_v[l]
        ec = ec * s + t
        lde = lde + jnp.sum(jnp.where(emask, _log_sigmoid(ps), 0.0))
    ze_ref[0] = ec
    lde_ref[...] = jnp.zeros((1, 1, 128), f32) + lde


def kernel(inp_node_features, inp_adj_features, noise_node, noise_edge,
           rgcn_Wself0, rgcn_Wrel0, rgcn_Wself1, rgcn_Wrel1, rgcn_Wself2,
           rgcn_Wrel2, node_Ws, node_bs, node_Wt, node_bt, edge_Ws, edge_bs,
           edge_Wt, edge_bt, rows, cols):
    f32 = jnp.float32

    xT = jnp.transpose(inp_node_features, (0, 2, 1))
    adjT = jnp.transpose(inp_adj_features, (0, 1, 3, 2))
    nnT = jnp.transpose(noise_node, (0, 2, 1))
    neP = jnp.pad(noise_edge, ((0, 0), (0, EPAD - E), (0, 0)))
    neT = jnp.transpose(neP, (0, 2, 1))

    rows_p = jnp.pad(rows, (0, EPAD - E), constant_values=-1)
    cols_p = jnp.pad(cols, (0, EPAD - E), constant_values=-1)
    n_iota = jnp.arange(N, dtype=rows_p.dtype)[:, None]
    OCT = (cols_p[None, :] == n_iota).astype(f32)
    ORT = (rows_p[None, :] == n_iota).astype(f32)

    ws0 = rgcn_Wself0.T
    wr0 = jnp.transpose(rgcn_Wrel0, (0, 2, 1))
    ws1 = rgcn_Wself1.T
    wr1 = jnp.transpose(rgcn_Wrel1, (0, 2, 1))
    ws2 = rgcn_Wself2.T
    wr2 = jnp.transpose(rgcn_Wrel2, (0, 2, 1))

    nws = jnp.transpose(node_Ws, (0, 2, 1))
    nwt = jnp.transpose(node_Wt, (0, 2, 1))
    nbs = jnp.broadcast_to(node_bs[:, :, None], (NFLOW, ND, N))
    nbt = jnp.broadcast_to(node_bt[:, :, None], (NFLOW, ND, N))

    estop = jnp.transpose(edge_Ws[:, :NOUT, :], (0, 2, 1)).reshape(NFLOW * BD, NOUT)
    esbot = jnp.transpose(edge_Ws[:, NOUT:, :], (0, 2, 1)).reshape(NFLOW * BD, NOUT)
    ettop = jnp.transpose(edge_Wt[:, :NOUT, :], (0, 2, 1)).reshape(NFLOW * BD, NOUT)
    etbot = jnp.transpose(edge_Wt[:, NOUT:, :], (0, 2, 1)).reshape(NFLOW * BD, NOUT)
    ebs = jnp.broadcast_to(edge_bs[:, :, None], (NFLOW, BD, EPAD))
    ebt = jnp.broadcast_to(edge_bt[:, :, None], (NFLOW, BD, EPAD))

    def rep(shape):
        nd = len(shape)
        return pl.BlockSpec(shape, lambda b, _n=nd: (0,) * _n)

    per_b = lambda shape: pl.BlockSpec((1,) + shape[1:], lambda b: (b,) + (0,) * (len(shape) - 1))

    in_specs = [
        per_b((B, ND, N)),
        per_b((B, BD, N, N)),
        per_b((B, ND, N)),
        per_b((B, BD, EPAD)),
        rep((N, EPAD)),
        rep((N, EPAD)),
        rep((NHID, ND)), rep((BD, NHID, ND)),
        rep((NHID, NHID)), rep((BD, NHID, NHID)),
        rep((NOUT, NHID)), rep((BD, NOUT, NHID)),
        rep((NFLOW, ND, NOUT)), rep((NFLOW, ND, N)),
        rep((NFLOW, ND, NOUT)), rep((NFLOW, ND, N)),
        rep((NFLOW * BD, NOUT)), rep((NFLOW * BD, NOUT)),
        rep((NFLOW * BD, NOUT)), rep((NFLOW * BD, NOUT)),
        rep((NFLOW, BD, EPAD)), rep((NFLOW, BD, EPAD)),
    ]
    out_specs = [
        per_b((B, ND, N)),
        per_b((B, BD, EPAD)),
        pl.BlockSpec((1, 1, 128), lambda b: (b, 0, 0)),
        pl.BlockSpec((1, 1, 128), lambda b: (b, 0, 0)),
    ]
    out_shapes = [
        jax.ShapeDtypeStruct((B, ND, N), f32),
        jax.ShapeDtypeStruct((B, BD, EPAD), f32),
        jax.ShapeDtypeStruct((B, 1, 128), f32),
        jax.ShapeDtypeStruct((B, 1, 128), f32),
    ]

    znT, zeT, ldn, lde = pl.pallas_call(
        _fused_body,
        grid=(B,),
        in_specs=in_specs,
        out_specs=out_specs,
        out_shape=out_shapes,
        compiler_params=pltpu.CompilerParams(
            dimension_semantics=("arbitrary",),
        ),
    )(xT, adjT, nnT, neT, OCT, ORT,
      ws0, wr0, ws1, wr1, ws2, wr2,
      nws, nbs, nwt, nbt,
      estop, esbot, ettop, etbot, ebs, ebt)

    z_node = jnp.transpose(znT, (0, 2, 1)).reshape(B, N * ND)
    z_edge = jnp.transpose(zeT, (0, 2, 1))[:, :E, :].reshape(B, E * BD)
    return (z_node, z_edge, ldn[:, 0, 0], lde[:, 0, 0])
